# baseline (speedup 1.0000x reference)
import jax
import jax.numpy as jnp
from jax.experimental import pallas as pl
from jax.experimental.pallas import tpu as pltpu

_NUM_EXPERTS = 64
_TOP_K = 8


def _router_kernel(x_ref, wt_ref, logits_ref, w_ref, i_ref):
    x = x_ref[...]
    wt = wt_ref[...]
    logits = jnp.dot(x, wt, preferred_element_type=jnp.float32)
    logits_ref[...] = logits
    m = jnp.max(logits, axis=1, keepdims=True)
    e = jnp.exp(logits - m)
    s = jnp.sum(e, axis=1, keepdims=True)
    p = e / s
    iota = jax.lax.broadcasted_iota(jnp.int32, p.shape, 1)
    vals = []
    idxs = []
    cur = p
    for _ in range(_TOP_K):
        mx = jnp.max(cur, axis=1, keepdims=True)
        amx = jnp.min(jnp.where(cur == mx, iota, _NUM_EXPERTS),
                      axis=1, keepdims=True)
        vals.append(mx)
        idxs.append(amx)
        cur = jnp.where(iota == amx, -jnp.inf, cur)
    w = jnp.concatenate(vals, axis=1)
    idx = jnp.concatenate(idxs, axis=1)
    w = w / jnp.sum(w, axis=1, keepdims=True)
    w_ref[...] = w
    i_ref[...] = idx


def kernel(hidden_states, router_weight):
    b, s, h = hidden_states.shape
    ne = router_weight.shape[0]
    x = hidden_states.reshape(b * s, h)
    wt = router_weight.T
    total = b * s
    bt = 512
    grid = (total // bt,)
    out_shape = [
        jax.ShapeDtypeStruct((total, ne), jnp.float32),
        jax.ShapeDtypeStruct((total, _TOP_K), jnp.float32),
        jax.ShapeDtypeStruct((total, _TOP_K), jnp.int32),
    ]
    logits, w, idx = pl.pallas_call(
        _router_kernel,
        grid=grid,
        in_specs=[
            pl.BlockSpec((bt, h), lambda i: (i, 0)),
            pl.BlockSpec((h, ne), lambda i: (0, 0)),
        ],
        out_specs=[
            pl.BlockSpec((bt, ne), lambda i: (i, 0)),
            pl.BlockSpec((bt, _TOP_K), lambda i: (i, 0)),
            pl.BlockSpec((bt, _TOP_K), lambda i: (i, 0)),
        ],
        out_shape=out_shape,
    )(x, wt)
    return (w, idx, logits)


# speedup vs baseline: 1.1645x; 1.1645x over previous
import jax
import jax.numpy as jnp
from jax.experimental import pallas as pl
from jax.experimental.pallas import tpu as pltpu

_NUM_EXPERTS = 64
_TOP_K = 8


def _router_kernel(x_ref, wt_ref, logits_ref, w_ref, i_ref):
    x = x_ref[...]
    wt = wt_ref[...]
    logits = jnp.dot(x, wt, preferred_element_type=jnp.float32)
    logits_ref[...] = logits
    m = jnp.max(logits, axis=1, keepdims=True)
    e = jnp.exp(logits - m)
    s = jnp.sum(e, axis=1, keepdims=True)
    p = e / s
    iota = jax.lax.broadcasted_iota(jnp.int32, p.shape, 1)
    vals = []
    idxs = []
    cur = p
    for _ in range(_TOP_K):
        mx = jnp.max(cur, axis=1, keepdims=True)
        amx = jnp.min(jnp.where(cur == mx, iota, _NUM_EXPERTS),
                      axis=1, keepdims=True)
        vals.append(mx)
        idxs.append(amx)
        cur = jnp.where(iota == amx, -jnp.inf, cur)
    w = jnp.concatenate(vals, axis=1)
    idx = jnp.concatenate(idxs, axis=1)
    w = w / jnp.sum(w, axis=1, keepdims=True)
    w_ref[...] = w
    i_ref[...] = idx


def kernel(hidden_states, router_weight):
    b, s, h = hidden_states.shape
    ne = router_weight.shape[0]
    x = hidden_states.reshape(b * s, h)
    wt = router_weight.T
    total = b * s
    bt = 1024
    grid = (total // bt,)
    out_shape = [
        jax.ShapeDtypeStruct((total, ne), jnp.float32),
        jax.ShapeDtypeStruct((total, _TOP_K), jnp.float32),
        jax.ShapeDtypeStruct((total, _TOP_K), jnp.int32),
    ]
    logits, w, idx = pl.pallas_call(
        _router_kernel,
        grid=grid,
        in_specs=[
            pl.BlockSpec((bt, h), lambda i: (i, 0)),
            pl.BlockSpec((h, ne), lambda i: (0, 0)),
        ],
        out_specs=[
            pl.BlockSpec((bt, ne), lambda i: (i, 0)),
            pl.BlockSpec((bt, _TOP_K), lambda i: (i, 0)),
            pl.BlockSpec((bt, _TOP_K), lambda i: (i, 0)),
        ],
        out_shape=out_shape,
    )(x, wt)
    return (w, idx, logits)


# speedup vs baseline: 1.1911x; 1.0228x over previous
import jax
import jax.numpy as jnp
from jax.experimental import pallas as pl
from jax.experimental.pallas import tpu as pltpu

_NUM_EXPERTS = 64
_TOP_K = 8


def _router_kernel(x_ref, wt_ref, logits_ref, w_ref, i_ref):
    x = x_ref[...]
    wt = wt_ref[...]
    logits = jnp.dot(x, wt, preferred_element_type=jnp.float32)
    logits_ref[...] = logits
    m = jnp.max(logits, axis=1, keepdims=True)
    e = jnp.exp(logits - m)
    s = jnp.sum(e, axis=1, keepdims=True)
    p = e / s
    iota = jax.lax.broadcasted_iota(jnp.int32, p.shape, 1)
    vals = []
    idxs = []
    cur = p
    for _ in range(_TOP_K):
        mx = jnp.max(cur, axis=1, keepdims=True)
        amx = jnp.min(jnp.where(cur == mx, iota, _NUM_EXPERTS),
                      axis=1, keepdims=True)
        vals.append(mx)
        idxs.append(amx)
        cur = jnp.where(iota == amx, -jnp.inf, cur)
    w = jnp.concatenate(vals, axis=1)
    idx = jnp.concatenate(idxs, axis=1)
    w = w / jnp.sum(w, axis=1, keepdims=True)
    w_ref[...] = w
    i_ref[...] = idx


def kernel(hidden_states, router_weight):
    b, s, h = hidden_states.shape
    ne = router_weight.shape[0]
    x = hidden_states.reshape(b * s, h)
    wt = router_weight.T
    total = b * s
    bt = 2048
    grid = (total // bt,)
    out_shape = [
        jax.ShapeDtypeStruct((total, ne), jnp.float32),
        jax.ShapeDtypeStruct((total, _TOP_K), jnp.float32),
        jax.ShapeDtypeStruct((total, _TOP_K), jnp.int32),
    ]
    logits, w, idx = pl.pallas_call(
        _router_kernel,
        grid=grid,
        in_specs=[
            pl.BlockSpec((bt, h), lambda i: (i, 0)),
            pl.BlockSpec((h, ne), lambda i: (0, 0)),
        ],
        out_specs=[
            pl.BlockSpec((bt, ne), lambda i: (i, 0)),
            pl.BlockSpec((bt, _TOP_K), lambda i: (i, 0)),
            pl.BlockSpec((bt, _TOP_K), lambda i: (i, 0)),
        ],
        out_shape=out_shape,
    )(x, wt)
    return (w, idx, logits)
